# baseline (speedup 1.0000x reference)
import functools

import jax
import jax.numpy as jnp
from jax import lax
from jax.experimental import pallas as pl
from jax.experimental.pallas import tpu as pltpu
from jax.experimental.pallas import tpu_sc as plsc

_N, _E, _F, _D, _C = 10000, 320000, 128, 32, 2
_BN_EPS = 1e-5
_NPAD = 10240
_NC, _NS = 2, 16
_NW = _NC * _NS
_CHUNK = 96
_CPT = 106
_CAP = _NW * _CPT * _CHUNK
_RPT = _NPAD // _NS


def _sc_segment_sum(h, srcp, dstp, zer, width):
    mesh = plsc.VectorSubcoreMesh(core_axis_name="c", subcore_axis_name="s")

    @functools.partial(
        pl.kernel,
        mesh=mesh,
        compiler_params=pltpu.CompilerParams(use_tc_tiling_on_sc=False),
        out_type=jax.ShapeDtypeStruct((_NC * _NPAD, width), jnp.float32),
        scratch_types=[
            pltpu.VMEM((_CPT, _CHUNK), jnp.int32),
            pltpu.VMEM((_CPT, _CHUNK), jnp.int32),
            pltpu.VMEM((_CHUNK, width), jnp.float32),
            pltpu.VMEM((_CHUNK, width), jnp.float32),
            pltpu.VMEM_SHARED((_NPAD, width), jnp.float32),
            pltpu.SemaphoreType.DMA,
            pltpu.SemaphoreType.DMA,
        ],
    )
    def k(h_hbm, srcp_hbm, dstp_hbm, zer_hbm, out_hbm,
          src_v, dst_v, rows0, rows1, acc_sh, sem0, sem1):
        c = lax.axis_index("c")
        s = lax.axis_index("s")
        wid = c * _NS + s
        r0 = s * _RPT
        pltpu.sync_copy(zer_hbm.at[pl.ds(r0, _RPT)],
                        acc_sh.at[pl.ds(r0, _RPT)])
        pltpu.sync_copy(srcp_hbm.at[wid], src_v)
        pltpu.sync_copy(dstp_hbm.at[wid], dst_v)
        plsc.subcore_barrier()

        bufs = (rows0, rows1)
        sems = (sem0, sem1)
        for b in range(2):
            pltpu.async_copy(h_hbm.at[src_v.at[b]], bufs[b], sems[b])

        def body(i, carry):
            j = 2 * i
            for b in range(2):
                pltpu.make_async_copy(h_hbm.at[src_v.at[0]],
                                      bufs[b], sems[b]).wait()
                pltpu.sync_copy(bufs[b], acc_sh.at[dst_v.at[j + b]], add=True)
                pltpu.async_copy(h_hbm.at[src_v.at[j + 2 + b]],
                                 bufs[b], sems[b])
            return carry

        lax.fori_loop(0, _CPT // 2 - 1, body, 0)
        for b in range(2):
            pltpu.make_async_copy(h_hbm.at[src_v.at[0]],
                                  bufs[b], sems[b]).wait()
            pltpu.sync_copy(bufs[b], acc_sh.at[dst_v.at[_CPT - 2 + b]],
                            add=True)
        plsc.subcore_barrier()
        pltpu.sync_copy(acc_sh.at[pl.ds(r0, _RPT)],
                        out_hbm.at[pl.ds(c * _NPAD + r0, _RPT)])

    return k(h, srcp, dstp, zer)


_GRID = 8
_BR = _NPAD // _GRID

def _row_spec(width):
    return pl.BlockSpec((_BR, width), lambda i: (i, 0))

def _full_spec(a, b):
    return pl.BlockSpec((a, b), lambda i: (0, 0))

def _part_spec(width):
    return pl.BlockSpec((_NC, _BR, width), lambda i: (0, i, 0))

_INVSQ = 1.0 / (1.0 + _BN_EPS) ** 0.5


def _tc_layer(p, h, Wa, ba, Wb, bb, g, be, width):
    def body(p_ref, h_ref, wa_ref, ba_ref, wb_ref, bb_ref, g_ref, be_ref,
             o_ref):
        u = h_ref[...] + p_ref[0] + p_ref[1]
        t1 = jax.nn.relu(jnp.dot(u, wa_ref[...],
                                 preferred_element_type=jnp.float32)
                         + ba_ref[...])
        t = jnp.dot(t1, wb_ref[...],
                    preferred_element_type=jnp.float32) + bb_ref[...]
        o_ref[...] = jax.nn.relu(t) * (g_ref[...] * _INVSQ) + be_ref[...]
    return pl.pallas_call(
        body,
        grid=(_GRID,),
        in_specs=[_part_spec(width), _row_spec(width), _full_spec(width, _D),
                  _full_spec(1, _D), _full_spec(_D, _D), _full_spec(1, _D),
                  _full_spec(1, _D), _full_spec(1, _D)],
        out_specs=_row_spec(_D),
        out_shape=jax.ShapeDtypeStruct((_NPAD, _D), jnp.float32),
    )(p, h, Wa, ba, Wb, bb, g, be)


def _tc_head(p, h, Wa, ba, Wb, bb, g, be, Wf1, bf1, Wf2, bf2):
    def body(p_ref, h_ref, wa_ref, ba_ref, wb_ref, bb_ref, g_ref, be_ref,
             wf1_ref, bf1_ref, wf2_ref, bf2_ref, o_ref):
        u = h_ref[...] + p_ref[0] + p_ref[1]
        t1 = jax.nn.relu(jnp.dot(u, wa_ref[...],
                                 preferred_element_type=jnp.float32)
                         + ba_ref[...])
        t = jnp.dot(t1, wb_ref[...],
                    preferred_element_type=jnp.float32) + bb_ref[...]
        hh = jax.nn.relu(t) * (g_ref[...] * _INVSQ) + be_ref[...]
        hh = jax.nn.relu(jnp.dot(hh, wf1_ref[...],
                                 preferred_element_type=jnp.float32)
                         + bf1_ref[...])
        o_ref[...] = jnp.dot(hh, wf2_ref[...],
                             preferred_element_type=jnp.float32) + bf2_ref[...]
    return pl.pallas_call(
        body,
        grid=(_GRID,),
        in_specs=[_part_spec(_D), _row_spec(_D), _full_spec(_D, _D),
                  _full_spec(1, _D), _full_spec(_D, _D), _full_spec(1, _D),
                  _full_spec(1, _D), _full_spec(1, _D), _full_spec(_D, _D),
                  _full_spec(1, _D), _full_spec(_D, _C), _full_spec(1, _C)],
        out_specs=_row_spec(_C),
        out_shape=jax.ShapeDtypeStruct((_NPAD, _C), jnp.float32),
    )(p, h, Wa, ba, Wb, bb, g, be, Wf1, bf1, Wf2, bf2)


def kernel(x, edge_index, edge_attr, batch,
           W11, b11, W12, b12, g1, be1,
           W21, b21, W22, b22, g2, be2,
           W31, b31, W32, b32, g3, be3,
           Wf1, bf1, Wf2, bf2):
    src = edge_index[0]
    dst = edge_index[1]
    srcp = jnp.concatenate(
        [src, jnp.zeros((_CAP - _E,), jnp.int32)]).reshape(_NW, _CPT, _CHUNK)
    dstp = jnp.concatenate(
        [dst, jnp.full((_CAP - _E,), _N, jnp.int32)]).reshape(_NW, _CPT, _CHUNK)
    xp = jnp.pad(x, ((0, _NPAD - _N), (0, 0)))
    zerF = jnp.zeros((_NPAD, _F), jnp.float32)
    zerD = jnp.zeros((_NPAD, _D), jnp.float32)

    r = lambda v: v.reshape(1, -1)
    pF = lambda p: p.reshape(_NC, _NPAD, _F)
    pD = lambda p: p.reshape(_NC, _NPAD, _D)

    p1 = _sc_segment_sum(xp, srcp, dstp, zerF, _F)
    h1 = _tc_layer(pF(p1), xp, W11, r(b11), W12, r(b12), r(g1), r(be1), _F)
    p2 = _sc_segment_sum(h1, srcp, dstp, zerD, _D)
    h2 = _tc_layer(pD(p2), h1, W21, r(b21), W22, r(b22), r(g2), r(be2), _D)
    p3 = _sc_segment_sum(h2, srcp, dstp, zerD, _D)
    out = _tc_head(pD(p3), h2, W31, r(b31), W32, r(b32), r(g3), r(be3),
                   Wf1, r(bf1), Wf2, r(bf2))
    return out[:_N]


# speedup vs baseline: 1.5312x; 1.5312x over previous
import functools

import jax
import jax.numpy as jnp
from jax import lax
from jax.experimental import pallas as pl
from jax.experimental.pallas import tpu as pltpu
from jax.experimental.pallas import tpu_sc as plsc

_N, _E, _F, _D, _C = 10000, 320000, 128, 32, 2
_BN_EPS = 1e-5
_NPAD = 10240
_NC, _NS = 2, 16
_NW = _NC * _NS
_RPT = _NPAD // _NS

_FAST = 1
_CH1, _CF1, _CS1 = 64, 238, 76
_CH2, _CF2, _CS2 = 96, 136, 74


def _sc_segment_sum(h, srcp, dstp, zer, width, chunk, cpt_f, cpt_s):
    mesh = plsc.VectorSubcoreMesh(core_axis_name="c", subcore_axis_name="s")

    @functools.partial(
        pl.kernel,
        mesh=mesh,
        compiler_params=pltpu.CompilerParams(use_tc_tiling_on_sc=False),
        out_type=jax.ShapeDtypeStruct((_NC * _NPAD, width), jnp.float32),
        scratch_types=[
            pltpu.VMEM((cpt_f, chunk), jnp.int32),
            pltpu.VMEM((cpt_f, chunk), jnp.int32),
            pltpu.VMEM((chunk, width), jnp.float32),
            pltpu.VMEM((chunk, width), jnp.float32),
            pltpu.VMEM_SHARED((_NPAD, width), jnp.float32),
            pltpu.SemaphoreType.DMA,
            pltpu.SemaphoreType.DMA,
        ],
    )
    def k(h_hbm, srcp_hbm, dstp_hbm, zer_hbm, out_hbm,
          src_v, dst_v, rows0, rows1, acc_sh, sem0, sem1):
        c = lax.axis_index("c")
        s = lax.axis_index("s")
        slab = jnp.where(c == _FAST, s, _NS + s)
        mycpt = jnp.where(c == _FAST, cpt_f, cpt_s)
        r0 = s * _RPT
        pltpu.sync_copy(zer_hbm.at[pl.ds(r0, _RPT)],
                        acc_sh.at[pl.ds(r0, _RPT)])
        pltpu.sync_copy(srcp_hbm.at[slab], src_v)
        pltpu.sync_copy(dstp_hbm.at[slab], dst_v)
        plsc.subcore_barrier()

        bufs = (rows0, rows1)
        sems = (sem0, sem1)
        for b in range(2):
            pltpu.async_copy(h_hbm.at[src_v.at[b]], bufs[b], sems[b])

        def body(i, carry):
            j = 2 * i
            for b in range(2):
                pltpu.make_async_copy(h_hbm.at[src_v.at[0]],
                                      bufs[b], sems[b]).wait()
                pltpu.sync_copy(bufs[b], acc_sh.at[dst_v.at[j + b]], add=True)
                pltpu.async_copy(h_hbm.at[src_v.at[j + 2 + b]],
                                 bufs[b], sems[b])
            return carry

        lax.fori_loop(0, mycpt // 2 - 1, body, 0)
        for b in range(2):
            pltpu.make_async_copy(h_hbm.at[src_v.at[0]],
                                  bufs[b], sems[b]).wait()
            pltpu.sync_copy(bufs[b], acc_sh.at[dst_v.at[mycpt - 2 + b]],
                            add=True)
        plsc.subcore_barrier()
        pltpu.sync_copy(acc_sh.at[pl.ds(r0, _RPT)],
                        out_hbm.at[pl.ds(c * _NPAD + r0, _RPT)])

    return k(h, srcp, dstp, zer)


_GRID = 8
_BR = _NPAD // _GRID

def _row_spec(width):
    return pl.BlockSpec((_BR, width), lambda i: (i, 0))

def _full_spec(a, b):
    return pl.BlockSpec((a, b), lambda i: (0, 0))

def _part_spec(width):
    return pl.BlockSpec((_NC, _BR, width), lambda i: (0, i, 0))

_INVSQ = 1.0 / (1.0 + _BN_EPS) ** 0.5


def _tc_layer(p, h, Wa, ba, Wb, bb, g, be, width):
    def body(p_ref, h_ref, wa_ref, ba_ref, wb_ref, bb_ref, g_ref, be_ref,
             o_ref):
        u = h_ref[...] + p_ref[0] + p_ref[1]
        t1 = jax.nn.relu(jnp.dot(u, wa_ref[...],
                                 preferred_element_type=jnp.float32)
                         + ba_ref[...])
        t = jnp.dot(t1, wb_ref[...],
                    preferred_element_type=jnp.float32) + bb_ref[...]
        o_ref[...] = jax.nn.relu(t) * (g_ref[...] * _INVSQ) + be_ref[...]
    return pl.pallas_call(
        body,
        grid=(_GRID,),
        in_specs=[_part_spec(width), _row_spec(width), _full_spec(width, _D),
                  _full_spec(1, _D), _full_spec(_D, _D), _full_spec(1, _D),
                  _full_spec(1, _D), _full_spec(1, _D)],
        out_specs=_row_spec(_D),
        out_shape=jax.ShapeDtypeStruct((_NPAD, _D), jnp.float32),
    )(p, h, Wa, ba, Wb, bb, g, be)


def _tc_head(p, h, Wa, ba, Wb, bb, g, be, Wf1, bf1, Wf2, bf2):
    def body(p_ref, h_ref, wa_ref, ba_ref, wb_ref, bb_ref, g_ref, be_ref,
             wf1_ref, bf1_ref, wf2_ref, bf2_ref, o_ref):
        u = h_ref[...] + p_ref[0] + p_ref[1]
        t1 = jax.nn.relu(jnp.dot(u, wa_ref[...],
                                 preferred_element_type=jnp.float32)
                         + ba_ref[...])
        t = jnp.dot(t1, wb_ref[...],
                    preferred_element_type=jnp.float32) + bb_ref[...]
        hh = jax.nn.relu(t) * (g_ref[...] * _INVSQ) + be_ref[...]
        hh = jax.nn.relu(jnp.dot(hh, wf1_ref[...],
                                 preferred_element_type=jnp.float32)
                         + bf1_ref[...])
        o_ref[...] = jnp.dot(hh, wf2_ref[...],
                             preferred_element_type=jnp.float32) + bf2_ref[...]
    return pl.pallas_call(
        body,
        grid=(_GRID,),
        in_specs=[_part_spec(_D), _row_spec(_D), _full_spec(_D, _D),
                  _full_spec(1, _D), _full_spec(_D, _D), _full_spec(1, _D),
                  _full_spec(1, _D), _full_spec(1, _D), _full_spec(_D, _D),
                  _full_spec(1, _D), _full_spec(_D, _C), _full_spec(1, _C)],
        out_specs=_row_spec(_C),
        out_shape=jax.ShapeDtypeStruct((_NPAD, _C), jnp.float32),
    )(p, h, Wa, ba, Wb, bb, g, be, Wf1, bf1, Wf2, bf2)


def kernel(x, edge_index, edge_attr, batch,
           W11, b11, W12, b12, g1, be1,
           W21, b21, W22, b22, g2, be2,
           W31, b31, W32, b32, g3, be3,
           Wf1, bf1, Wf2, bf2):
    src = edge_index[0]
    dst = edge_index[1]

    def pack(idx, padval, chunk, cpt_f, cpt_s):
        nf = _NS * cpt_f * chunk
        ns = _NS * cpt_s * chunk
        f = idx[:nf].reshape(_NS, cpt_f, chunk)
        tail = jnp.concatenate(
            [idx[nf:], jnp.full((ns - (_E - nf),), padval, jnp.int32)])
        sl = jnp.pad(tail.reshape(_NS, cpt_s, chunk),
                     ((0, 0), (0, cpt_f - cpt_s), (0, 0)),
                     constant_values=padval)
        return jnp.concatenate([f, sl], axis=0)

    srcp1 = pack(src, 0, _CH1, _CF1, _CS1)
    dstp1 = pack(dst, _N, _CH1, _CF1, _CS1)
    srcp2 = pack(src, 0, _CH2, _CF2, _CS2)
    dstp2 = pack(dst, _N, _CH2, _CF2, _CS2)
    xp = jnp.pad(x, ((0, _NPAD - _N), (0, 0)))
    zerF = jnp.zeros((_NPAD, _F), jnp.float32)
    zerD = jnp.zeros((_NPAD, _D), jnp.float32)

    r = lambda v: v.reshape(1, -1)
    pF = lambda p: p.reshape(_NC, _NPAD, _F)
    pD = lambda p: p.reshape(_NC, _NPAD, _D)

    p1 = _sc_segment_sum(xp, srcp1, dstp1, zerF, _F, _CH1, _CF1, _CS1)
    h1 = _tc_layer(pF(p1), xp, W11, r(b11), W12, r(b12), r(g1), r(be1), _F)
    p2 = _sc_segment_sum(h1, srcp2, dstp2, zerD, _D, _CH2, _CF2, _CS2)
    h2 = _tc_layer(pD(p2), h1, W21, r(b21), W22, r(b22), r(g2), r(be2), _D)
    p3 = _sc_segment_sum(h2, srcp2, dstp2, zerD, _D, _CH2, _CF2, _CS2)
    out = _tc_head(pD(p3), h2, W31, r(b31), W32, r(b32), r(g3), r(be3),
                   Wf1, r(bf1), Wf2, r(bf2))
    return out[:_N]


# speedup vs baseline: 1.7921x; 1.1703x over previous
import functools

import jax
import jax.numpy as jnp
from jax import lax
from jax.experimental import pallas as pl
from jax.experimental.pallas import tpu as pltpu
from jax.experimental.pallas import tpu_sc as plsc

_N, _E, _F, _D, _C = 10000, 320000, 128, 32, 2
_BN_EPS = 1e-5
_NC, _NS = 2, 16
_RPT = _N // _NS
_CHUNK = 80

_FAST = 1
_CF1, _CS1 = 170, 80
_CF2, _CS2 = 152, 98


def _sc_segment_sum(h, srcp, dstp, zer, width, cpt_f, cpt_s):
    mesh = plsc.VectorSubcoreMesh(core_axis_name="c", subcore_axis_name="s")

    @functools.partial(
        pl.kernel,
        mesh=mesh,
        compiler_params=pltpu.CompilerParams(use_tc_tiling_on_sc=False),
        out_type=jax.ShapeDtypeStruct((_NC * _N, width), jnp.float32),
        scratch_types=[
            pltpu.VMEM((cpt_f, _CHUNK), jnp.int32),
            pltpu.VMEM((cpt_f, _CHUNK), jnp.int32),
            pltpu.VMEM((_CHUNK, width), jnp.float32),
            pltpu.VMEM((_CHUNK, width), jnp.float32),
            pltpu.VMEM_SHARED((_N, width), jnp.float32),
            pltpu.SemaphoreType.DMA,
            pltpu.SemaphoreType.DMA,
        ],
    )
    def k(h_hbm, srcp_hbm, dstp_hbm, zer_hbm, out_hbm,
          src_v, dst_v, rows0, rows1, acc_sh, sem0, sem1):
        c = lax.axis_index("c")
        s = lax.axis_index("s")
        fast = c == _FAST
        mycpt = jnp.where(fast, cpt_f, cpt_s)
        r0 = s * _RPT
        pltpu.sync_copy(zer_hbm, acc_sh.at[pl.ds(r0, _RPT)])

        @pl.when(fast)
        def _():
            pltpu.sync_copy(srcp_hbm.at[s, pl.ds(0, cpt_f)], src_v)
            pltpu.sync_copy(dstp_hbm.at[s, pl.ds(0, cpt_f)], dst_v)

        @pl.when(jnp.logical_not(fast))
        def _():
            pltpu.sync_copy(srcp_hbm.at[s, pl.ds(cpt_f, cpt_s)],
                            src_v.at[pl.ds(0, cpt_s)])
            pltpu.sync_copy(dstp_hbm.at[s, pl.ds(cpt_f, cpt_s)],
                            dst_v.at[pl.ds(0, cpt_s)])

        plsc.subcore_barrier()

        bufs = (rows0, rows1)
        sems = (sem0, sem1)
        for b in range(2):
            pltpu.async_copy(h_hbm.at[src_v.at[b]], bufs[b], sems[b])

        def body(i, carry):
            j = 2 * i
            for b in range(2):
                pltpu.make_async_copy(h_hbm.at[src_v.at[0]],
                                      bufs[b], sems[b]).wait()
                pltpu.sync_copy(bufs[b], acc_sh.at[dst_v.at[j + b]], add=True)
                pltpu.async_copy(h_hbm.at[src_v.at[j + 2 + b]],
                                 bufs[b], sems[b])
            return carry

        lax.fori_loop(0, mycpt // 2 - 1, body, 0)
        for b in range(2):
            pltpu.make_async_copy(h_hbm.at[src_v.at[0]],
                                  bufs[b], sems[b]).wait()
            pltpu.sync_copy(bufs[b], acc_sh.at[dst_v.at[mycpt - 2 + b]],
                            add=True)
        plsc.subcore_barrier()
        pltpu.sync_copy(acc_sh.at[pl.ds(r0, _RPT)],
                        out_hbm.at[pl.ds(c * _N + r0, _RPT)])

    return k(h, srcp, dstp, zer)


_GRID = 10
_BR = _N // _GRID

def _row_spec(width):
    return pl.BlockSpec((_BR, width), lambda i: (i, 0))

def _p1_spec(width):
    return pl.BlockSpec((_BR, width), lambda i: (i + _GRID, 0))

def _full_spec(a, b):
    return pl.BlockSpec((a, b), lambda i: (0, 0))

_INVSQ = 1.0 / (1.0 + _BN_EPS) ** 0.5


def _tc_layer(p, h, Wa, ba, Wb, bb, g, be, width):
    def body(p0_ref, p1_ref, h_ref, wa_ref, ba_ref, wb_ref, bb_ref, g_ref,
             be_ref, o_ref):
        u = h_ref[...] + p0_ref[...] + p1_ref[...]
        t1 = jax.nn.relu(jnp.dot(u, wa_ref[...],
                                 preferred_element_type=jnp.float32)
                         + ba_ref[...])
        t = jnp.dot(t1, wb_ref[...],
                    preferred_element_type=jnp.float32) + bb_ref[...]
        o_ref[...] = jax.nn.relu(t) * (g_ref[...] * _INVSQ) + be_ref[...]
    return pl.pallas_call(
        body,
        grid=(_GRID,),
        in_specs=[_row_spec(width), _p1_spec(width), _row_spec(width),
                  _full_spec(width, _D), _full_spec(1, _D),
                  _full_spec(_D, _D), _full_spec(1, _D), _full_spec(1, _D),
                  _full_spec(1, _D)],
        out_specs=_row_spec(_D),
        out_shape=jax.ShapeDtypeStruct((_N, _D), jnp.float32),
    )(p, p, h, Wa, ba, Wb, bb, g, be)


def _tc_head(p, h, Wa, ba, Wb, bb, g, be, Wf1, bf1, Wf2, bf2):
    def body(p0_ref, p1_ref, h_ref, wa_ref, ba_ref, wb_ref, bb_ref, g_ref,
             be_ref, wf1_ref, bf1_ref, wf2_ref, bf2_ref, o_ref):
        u = h_ref[...] + p0_ref[...] + p1_ref[...]
        t1 = jax.nn.relu(jnp.dot(u, wa_ref[...],
                                 preferred_element_type=jnp.float32)
                         + ba_ref[...])
        t = jnp.dot(t1, wb_ref[...],
                    preferred_element_type=jnp.float32) + bb_ref[...]
        hh = jax.nn.relu(t) * (g_ref[...] * _INVSQ) + be_ref[...]
        hh = jax.nn.relu(jnp.dot(hh, wf1_ref[...],
                                 preferred_element_type=jnp.float32)
                         + bf1_ref[...])
        o_ref[...] = jnp.dot(hh, wf2_ref[...],
                             preferred_element_type=jnp.float32) + bf2_ref[...]
    return pl.pallas_call(
        body,
        grid=(_GRID,),
        in_specs=[_row_spec(_D), _p1_spec(_D), _row_spec(_D),
                  _full_spec(_D, _D), _full_spec(1, _D), _full_spec(_D, _D),
                  _full_spec(1, _D), _full_spec(1, _D), _full_spec(1, _D),
                  _full_spec(_D, _D), _full_spec(1, _D), _full_spec(_D, _C),
                  _full_spec(1, _C)],
        out_specs=_row_spec(_C),
        out_shape=jax.ShapeDtypeStruct((_N, _C), jnp.float32),
    )(p, p, h, Wa, ba, Wb, bb, g, be, Wf1, bf1, Wf2, bf2)


def kernel(x, edge_index, edge_attr, batch,
           W11, b11, W12, b12, g1, be1,
           W21, b21, W22, b22, g2, be2,
           W31, b31, W32, b32, g3, be3,
           Wf1, bf1, Wf2, bf2):
    srcp = edge_index[0].reshape(_NS, 250, _CHUNK)
    dstp = edge_index[1].reshape(_NS, 250, _CHUNK)
    zerF = jnp.zeros((_RPT, _F), jnp.float32)
    zerD = jnp.zeros((_RPT, _D), jnp.float32)

    r = lambda v: v.reshape(1, -1)

    p1 = _sc_segment_sum(x, srcp, dstp, zerF, _F, _CF1, _CS1)
    h1 = _tc_layer(p1, x, W11, r(b11), W12, r(b12), r(g1), r(be1), _F)
    p2 = _sc_segment_sum(h1, srcp, dstp, zerD, _D, _CF2, _CS2)
    h2 = _tc_layer(p2, h1, W21, r(b21), W22, r(b22), r(g2), r(be2), _D)
    p3 = _sc_segment_sum(h2, srcp, dstp, zerD, _D, _CF2, _CS2)
    out = _tc_head(p3, h2, W31, r(b31), W32, r(b32), r(g3), r(be3),
                   Wf1, r(bf1), Wf2, r(bf2))
    return out


# speedup vs baseline: 2.1109x; 1.1779x over previous
import functools

import jax
import jax.numpy as jnp
from jax import lax
from jax.experimental import pallas as pl
from jax.experimental.pallas import tpu as pltpu
from jax.experimental.pallas import tpu_sc as plsc

_N, _E, _F, _D, _C = 10000, 320000, 128, 32, 2
_BN_EPS = 1e-5
_NC, _NS = 2, 16
_RPT = _N // _NS
_CHUNK = 80

_FAST = 1
_CF1, _CS1 = 134, 116
_CF2, _CS2 = 130, 120


def _sc_segment_sum(h, srcp, dstp, zer, width, cpt_f, cpt_s):
    mesh = plsc.VectorSubcoreMesh(core_axis_name="c", subcore_axis_name="s")

    @functools.partial(
        pl.kernel,
        mesh=mesh,
        compiler_params=pltpu.CompilerParams(use_tc_tiling_on_sc=False),
        out_type=jax.ShapeDtypeStruct((_NC * _N, _F), jnp.float32),
        scratch_types=[
            pltpu.VMEM((cpt_f, _CHUNK), jnp.int32),
            pltpu.VMEM((cpt_f, _CHUNK), jnp.int32),
            pltpu.VMEM((_CHUNK, width), jnp.float32),
            pltpu.VMEM((_CHUNK, width), jnp.float32),
            pltpu.VMEM_SHARED((_N, width), jnp.float32),
            pltpu.SemaphoreType.DMA,
            pltpu.SemaphoreType.DMA,
        ],
    )
    def k(h_hbm, srcp_hbm, dstp_hbm, zer_hbm, out_hbm,
          src_v, dst_v, rows0, rows1, acc_sh, sem0, sem1):
        c = lax.axis_index("c")
        s = lax.axis_index("s")
        fast = c == _FAST
        mycpt = jnp.where(fast, cpt_f, cpt_s)
        r0 = s * _RPT
        pltpu.sync_copy(zer_hbm, acc_sh.at[pl.ds(r0, _RPT)])

        @pl.when(fast)
        def _():
            pltpu.sync_copy(srcp_hbm.at[s, pl.ds(0, cpt_f)], src_v)
            pltpu.sync_copy(dstp_hbm.at[s, pl.ds(0, cpt_f)], dst_v)

        @pl.when(jnp.logical_not(fast))
        def _():
            pltpu.sync_copy(srcp_hbm.at[s, pl.ds(cpt_f, cpt_s)],
                            src_v.at[pl.ds(0, cpt_s)])
            pltpu.sync_copy(dstp_hbm.at[s, pl.ds(cpt_f, cpt_s)],
                            dst_v.at[pl.ds(0, cpt_s)])

        plsc.subcore_barrier()

        bufs = (rows0, rows1)
        sems = (sem0, sem1)
        for b in range(2):
            pltpu.async_copy(h_hbm.at[src_v.at[b]], bufs[b], sems[b])

        def body(i, carry):
            j = 2 * i
            for b in range(2):
                pltpu.make_async_copy(h_hbm.at[src_v.at[0]],
                                      bufs[b], sems[b]).wait()
                pltpu.sync_copy(bufs[b], acc_sh.at[dst_v.at[j + b]], add=True)
                pltpu.async_copy(h_hbm.at[src_v.at[j + 2 + b]],
                                 bufs[b], sems[b])
            return carry

        lax.fori_loop(0, mycpt // 2 - 1, body, 0)
        for b in range(2):
            pltpu.make_async_copy(h_hbm.at[src_v.at[0]],
                                  bufs[b], sems[b]).wait()
            pltpu.sync_copy(bufs[b], acc_sh.at[dst_v.at[mycpt - 2 + b]],
                            add=True)
        plsc.subcore_barrier()
        pltpu.sync_copy(acc_sh.at[pl.ds(r0, _RPT)],
                        out_hbm.at[pl.ds(c * _N + r0, _RPT), pl.ds(0, width)])

    return k(h, srcp, dstp, zer)


_GRID = 10
_BR = _N // _GRID

def _row_spec(width):
    return pl.BlockSpec((_BR, width), lambda i: (i, 0))

def _p1_spec(width):
    return pl.BlockSpec((_BR, width), lambda i: (i + _GRID, 0))

def _full_spec(a, b):
    return pl.BlockSpec((a, b), lambda i: (0, 0))

_INVSQ = 1.0 / (1.0 + _BN_EPS) ** 0.5


def _tc_layer(p, h, Wa, ba, Wb, bb, g, be, width):
    def body(p0_ref, p1_ref, h_ref, wa_ref, ba_ref, wb_ref, bb_ref, g_ref,
             be_ref, o_ref):
        u = h_ref[...] + p0_ref[:, :width] + p1_ref[:, :width]
        t1 = jax.nn.relu(jnp.dot(u, wa_ref[...],
                                 preferred_element_type=jnp.float32)
                         + ba_ref[...])
        t = jnp.dot(t1, wb_ref[...],
                    preferred_element_type=jnp.float32) + bb_ref[...]
        o_ref[...] = jax.nn.relu(t) * (g_ref[...] * _INVSQ) + be_ref[...]
    return pl.pallas_call(
        body,
        grid=(_GRID,),
        in_specs=[_row_spec(_F), _p1_spec(_F), _row_spec(width),
                  _full_spec(width, _D), _full_spec(1, _D),
                  _full_spec(_D, _D), _full_spec(1, _D), _full_spec(1, _D),
                  _full_spec(1, _D)],
        out_specs=_row_spec(_D),
        out_shape=jax.ShapeDtypeStruct((_N, _D), jnp.float32),
    )(p, p, h, Wa, ba, Wb, bb, g, be)


def _tc_head(p, h, Wa, ba, Wb, bb, g, be, Wf1, bf1, Wf2, bf2):
    def body(p0_ref, p1_ref, h_ref, wa_ref, ba_ref, wb_ref, bb_ref, g_ref,
             be_ref, wf1_ref, bf1_ref, wf2_ref, bf2_ref, o_ref):
        u = h_ref[...] + p0_ref[:, :_D] + p1_ref[:, :_D]
        t1 = jax.nn.relu(jnp.dot(u, wa_ref[...],
                                 preferred_element_type=jnp.float32)
                         + ba_ref[...])
        t = jnp.dot(t1, wb_ref[...],
                    preferred_element_type=jnp.float32) + bb_ref[...]
        hh = jax.nn.relu(t) * (g_ref[...] * _INVSQ) + be_ref[...]
        hh = jax.nn.relu(jnp.dot(hh, wf1_ref[...],
                                 preferred_element_type=jnp.float32)
                         + bf1_ref[...])
        o_ref[...] = jnp.dot(hh, wf2_ref[...],
                             preferred_element_type=jnp.float32) + bf2_ref[...]
    return pl.pallas_call(
        body,
        grid=(_GRID,),
        in_specs=[_row_spec(_F), _p1_spec(_F), _row_spec(_D),
                  _full_spec(_D, _D), _full_spec(1, _D), _full_spec(_D, _D),
                  _full_spec(1, _D), _full_spec(1, _D), _full_spec(1, _D),
                  _full_spec(_D, _D), _full_spec(1, _D), _full_spec(_D, _C),
                  _full_spec(1, _C)],
        out_specs=_row_spec(_C),
        out_shape=jax.ShapeDtypeStruct((_N, _C), jnp.float32),
    )(p, p, h, Wa, ba, Wb, bb, g, be, Wf1, bf1, Wf2, bf2)


def kernel(x, edge_index, edge_attr, batch,
           W11, b11, W12, b12, g1, be1,
           W21, b21, W22, b22, g2, be2,
           W31, b31, W32, b32, g3, be3,
           Wf1, bf1, Wf2, bf2):
    srcp = edge_index[0].reshape(_NS, 250, _CHUNK)
    dstp = edge_index[1].reshape(_NS, 250, _CHUNK)
    zerF = jnp.zeros((_RPT, _F), jnp.float32)
    zerD = jnp.zeros((_RPT, _D), jnp.float32)

    r = lambda v: v.reshape(1, -1)

    p1 = _sc_segment_sum(x, srcp, dstp, zerF, _F, _CF1, _CS1)
    h1 = _tc_layer(p1, x, W11, r(b11), W12, r(b12), r(g1), r(be1), _F)
    p2 = _sc_segment_sum(h1, srcp, dstp, zerD, _D, _CF2, _CS2)
    h2 = _tc_layer(p2, h1, W21, r(b21), W22, r(b22), r(g2), r(be2), _D)
    p3 = _sc_segment_sum(h2, srcp, dstp, zerD, _D, _CF2, _CS2)
    out = _tc_head(p3, h2, W31, r(b31), W32, r(b32), r(g3), r(be3),
                   Wf1, r(bf1), Wf2, r(bf2))
    return out


# speedup vs baseline: 2.2099x; 1.0469x over previous
import functools

import jax
import jax.numpy as jnp
from jax import lax
from jax.experimental import pallas as pl
from jax.experimental.pallas import tpu as pltpu
from jax.experimental.pallas import tpu_sc as plsc

_N, _E, _F, _D, _C = 10000, 320000, 128, 32, 2
_BN_EPS = 1e-5
_NC, _NS = 2, 16
_RPT = _N // _NS
_CHUNK = 80

_FAST = 1
_CF1, _CS1 = 126, 124
_CF2, _CS2 = 126, 124


def _sc_segment_sum(h, srcp, dstp, zer, width, cpt_f, cpt_s):
    mesh = plsc.VectorSubcoreMesh(core_axis_name="c", subcore_axis_name="s")

    @functools.partial(
        pl.kernel,
        mesh=mesh,
        compiler_params=pltpu.CompilerParams(use_tc_tiling_on_sc=False),
        out_type=jax.ShapeDtypeStruct((_NC * _N, _F), jnp.float32),
        scratch_types=[
            pltpu.VMEM((cpt_f, _CHUNK), jnp.int32),
            pltpu.VMEM((cpt_f, _CHUNK), jnp.int32),
            pltpu.VMEM((_CHUNK, width), jnp.float32),
            pltpu.VMEM((_CHUNK, width), jnp.float32),
            pltpu.VMEM_SHARED((_N, width), jnp.float32),
            pltpu.SemaphoreType.DMA,
            pltpu.SemaphoreType.DMA,
        ],
    )
    def k(h_hbm, srcp_hbm, dstp_hbm, zer_hbm, out_hbm,
          src_v, dst_v, rows0, rows1, acc_sh, sem0, sem1):
        c = lax.axis_index("c")
        s = lax.axis_index("s")
        fast = c == _FAST
        mycpt = jnp.where(fast, cpt_f, cpt_s)
        r0 = s * _RPT
        pltpu.sync_copy(zer_hbm, acc_sh.at[pl.ds(r0, _RPT)])

        @pl.when(fast)
        def _():
            pltpu.sync_copy(srcp_hbm.at[s, pl.ds(0, cpt_f)], src_v)
            pltpu.sync_copy(dstp_hbm.at[s, pl.ds(0, cpt_f)], dst_v)

        @pl.when(jnp.logical_not(fast))
        def _():
            pltpu.sync_copy(srcp_hbm.at[s, pl.ds(cpt_f, cpt_s)],
                            src_v.at[pl.ds(0, cpt_s)])
            pltpu.sync_copy(dstp_hbm.at[s, pl.ds(cpt_f, cpt_s)],
                            dst_v.at[pl.ds(0, cpt_s)])

        plsc.subcore_barrier()

        bufs = (rows0, rows1)
        sems = (sem0, sem1)
        for b in range(2):
            pltpu.async_copy(h_hbm.at[src_v.at[b]], bufs[b], sems[b])

        def body(i, carry):
            j = 2 * i
            for b in range(2):
                pltpu.make_async_copy(h_hbm.at[src_v.at[0]],
                                      bufs[b], sems[b]).wait()
                pltpu.sync_copy(bufs[b], acc_sh.at[dst_v.at[j + b]], add=True)
                pltpu.async_copy(h_hbm.at[src_v.at[j + 2 + b]],
                                 bufs[b], sems[b])
            return carry

        lax.fori_loop(0, mycpt // 2 - 1, body, 0)
        for b in range(2):
            pltpu.make_async_copy(h_hbm.at[src_v.at[0]],
                                  bufs[b], sems[b]).wait()
            pltpu.sync_copy(bufs[b], acc_sh.at[dst_v.at[mycpt - 2 + b]],
                            add=True)
        plsc.subcore_barrier()
        pltpu.sync_copy(acc_sh.at[pl.ds(r0, _RPT)],
                        out_hbm.at[pl.ds(c * _N + r0, _RPT), pl.ds(0, width)])

    return k(h, srcp, dstp, zer)


_GRID = 5
_BR = _N // _GRID

def _row_spec(width):
    return pl.BlockSpec((_BR, width), lambda i: (i, 0))

def _p1_spec(width):
    return pl.BlockSpec((_BR, width), lambda i: (i + _GRID, 0))

def _full_spec(a, b):
    return pl.BlockSpec((a, b), lambda i: (0, 0))

_INVSQ = 1.0 / (1.0 + _BN_EPS) ** 0.5


def _tc_layer(p, h, Wa, ba, Wb, bb, g, be, width):
    def body(p0_ref, p1_ref, h_ref, wa_ref, ba_ref, wb_ref, bb_ref, g_ref,
             be_ref, o_ref):
        u = h_ref[...] + p0_ref[:, :width] + p1_ref[:, :width]
        t1 = jax.nn.relu(jnp.dot(u, wa_ref[...],
                                 preferred_element_type=jnp.float32)
                         + ba_ref[...])
        t = jnp.dot(t1, wb_ref[...],
                    preferred_element_type=jnp.float32) + bb_ref[...]
        o_ref[...] = jax.nn.relu(t) * (g_ref[...] * _INVSQ) + be_ref[...]
    return pl.pallas_call(
        body,
        grid=(_GRID,),
        in_specs=[_row_spec(_F), _p1_spec(_F), _row_spec(width),
                  _full_spec(width, _D), _full_spec(1, _D),
                  _full_spec(_D, _D), _full_spec(1, _D), _full_spec(1, _D),
                  _full_spec(1, _D)],
        out_specs=_row_spec(_D),
        out_shape=jax.ShapeDtypeStruct((_N, _D), jnp.float32),
    )(p, p, h, Wa, ba, Wb, bb, g, be)


def _tc_head(p, h, Wa, ba, Wb, bb, g, be, Wf1, bf1, Wf2, bf2):
    def body(p0_ref, p1_ref, h_ref, wa_ref, ba_ref, wb_ref, bb_ref, g_ref,
             be_ref, wf1_ref, bf1_ref, wf2_ref, bf2_ref, o_ref):
        u = h_ref[...] + p0_ref[:, :_D] + p1_ref[:, :_D]
        t1 = jax.nn.relu(jnp.dot(u, wa_ref[...],
                                 preferred_element_type=jnp.float32)
                         + ba_ref[...])
        t = jnp.dot(t1, wb_ref[...],
                    preferred_element_type=jnp.float32) + bb_ref[...]
        hh = jax.nn.relu(t) * (g_ref[...] * _INVSQ) + be_ref[...]
        hh = jax.nn.relu(jnp.dot(hh, wf1_ref[...],
                                 preferred_element_type=jnp.float32)
                         + bf1_ref[...])
        o_ref[...] = jnp.dot(hh, wf2_ref[...],
                             preferred_element_type=jnp.float32) + bf2_ref[...]
    return pl.pallas_call(
        body,
        grid=(_GRID,),
        in_specs=[_row_spec(_F), _p1_spec(_F), _row_spec(_D),
                  _full_spec(_D, _D), _full_spec(1, _D), _full_spec(_D, _D),
                  _full_spec(1, _D), _full_spec(1, _D), _full_spec(1, _D),
                  _full_spec(_D, _D), _full_spec(1, _D), _full_spec(_D, _C),
                  _full_spec(1, _C)],
        out_specs=_row_spec(_C),
        out_shape=jax.ShapeDtypeStruct((_N, _C), jnp.float32),
    )(p, p, h, Wa, ba, Wb, bb, g, be, Wf1, bf1, Wf2, bf2)


def kernel(x, edge_index, edge_attr, batch,
           W11, b11, W12, b12, g1, be1,
           W21, b21, W22, b22, g2, be2,
           W31, b31, W32, b32, g3, be3,
           Wf1, bf1, Wf2, bf2):
    srcp = edge_index[0].reshape(_NS, 250, _CHUNK)
    dstp = edge_index[1].reshape(_NS, 250, _CHUNK)
    zerF = jnp.zeros((_RPT, _F), jnp.float32)
    zerD = jnp.zeros((_RPT, _D), jnp.float32)

    r = lambda v: v.reshape(1, -1)

    p1 = _sc_segment_sum(x, srcp, dstp, zerF, _F, _CF1, _CS1)
    h1 = _tc_layer(p1, x, W11, r(b11), W12, r(b12), r(g1), r(be1), _F)
    p2 = _sc_segment_sum(h1, srcp, dstp, zerD, _D, _CF2, _CS2)
    h2 = _tc_layer(p2, h1, W21, r(b21), W22, r(b22), r(g2), r(be2), _D)
    p3 = _sc_segment_sum(h2, srcp, dstp, zerD, _D, _CF2, _CS2)
    out = _tc_head(p3, h2, W31, r(b31), W32, r(b32), r(g3), r(be3),
                   Wf1, r(bf1), Wf2, r(bf2))
    return out
